# baseline (speedup 1.0000x reference)
import functools

import jax
import jax.numpy as jnp
from jax.experimental import pallas as pl
from jax.experimental.pallas import tpu as pltpu


def _gelu(x):
    return x * jax.nn.sigmoid(1.5957691216057308 * x +
                              0.07135481627362622 * (x * x * x))


def _fused_block(x_ref, wpre_ref, wres_ref, wrt_ref, w1_ref, w2_ref,
                 wpost_ref, out_ref, carry_ref, wres16_ref, w116_ref,
                 w216_ref, wpost16_ref, *, lb):
    i = pl.program_id(0)
    f32 = jnp.float32
    bf16 = jnp.bfloat16

    @pl.when(i == 0)
    def _():
        carry_ref[...] = jnp.zeros_like(carry_ref)
        wres16_ref[...] = wres_ref[...].astype(bf16)
        w116_ref[...] = w1_ref[...].astype(bf16)
        w216_ref[...] = w2_ref[...].astype(bf16)
        wpost16_ref[...] = wpost_ref[...].astype(bf16)

    x = x_ref[0]
    h = _gelu(jnp.dot(x, wpre_ref[...], preferred_element_type=f32))
    h16 = h.astype(bf16)
    res = jnp.dot(h16, wres16_ref[...], preferred_element_type=f32)
    logits = jnp.dot(h, wrt_ref[...], preferred_element_type=f32)
    probs = jax.nn.sigmoid(logits)

    row = jax.lax.broadcasted_iota(jnp.int32, (lb, 1), 0)
    mask = (probs >= 0.5) | ((row == 0) & (i == 0))

    t16 = _gelu(jnp.dot(h16, w116_ref[...],
                              preferred_element_type=f32)).astype(bf16)
    mid = jnp.dot(t16, w216_ref[...], preferred_element_type=f32)
    yg = mid * probs

    rowf = row.astype(f32)
    c_col = jnp.where(mask, rowf, -1.0)
    rowi = jax.lax.broadcasted_iota(jnp.int32, (lb, lb), 0)
    colj = jax.lax.broadcasted_iota(jnp.int32, (lb, lb), 1)
    eye = (rowi == colj).astype(f32)
    c_row = jnp.dot(jnp.ones((1, lb), f32), eye * c_col,
                    preferred_element_type=f32)
    m_mat = jnp.where(colj <= rowi, jnp.broadcast_to(c_row, (lb, lb)), -1.0)
    b_col = jnp.max(m_mat, axis=1, keepdims=True)
    sel = (b_col == colj.astype(f32)).astype(f32)

    carry_row = carry_ref[7:8, :]
    up = jnp.dot(sel, yg, preferred_element_type=f32)
    up = up + jnp.where(b_col < 0.0, carry_row, 0.0)
    carry_ref[...] = up[lb - 8:, :]

    fused16 = (res + up).astype(bf16)
    out_ref[0] = _gelu(jnp.dot(fused16, wpost16_ref[...],
                                     preferred_element_type=f32))


def kernel(hidden_states, x_pack_kwargs, W_pre, W_res, w_router, W1, W2,
           W_post):
    del x_pack_kwargs
    B, L, D = hidden_states.shape
    d_ff = W1.shape[1]
    lb = 256
    wrt = w_router.reshape(D, 1)

    grid = (L // lb,)
    full = lambda a: pl.BlockSpec(a.shape, lambda i: (0,) * a.ndim)
    out = pl.pallas_call(
        functools.partial(_fused_block, lb=lb),
        grid=grid,
        in_specs=[
            pl.BlockSpec((1, lb, D), lambda i: (0, i, 0)),
            full(W_pre), full(W_res), full(wrt), full(W1), full(W2),
            full(W_post),
        ],
        out_specs=pl.BlockSpec((1, lb, D), lambda i: (0, i, 0)),
        out_shape=jax.ShapeDtypeStruct((B, L, D), jnp.float32),
        scratch_shapes=[
            pltpu.VMEM((8, D), jnp.float32),
            pltpu.VMEM((D, D), jnp.bfloat16),
            pltpu.VMEM((D, d_ff), jnp.bfloat16),
            pltpu.VMEM((d_ff, D), jnp.bfloat16),
            pltpu.VMEM((D, D), jnp.bfloat16),
        ],
        compiler_params=pltpu.CompilerParams(
            dimension_semantics=("arbitrary",)),
    )(hidden_states, W_pre, W_res, wrt, W1, W2, W_post)
    return out


# speedup vs baseline: 1.0819x; 1.0819x over previous
import functools

import jax
import jax.numpy as jnp
from jax.experimental import pallas as pl
from jax.experimental.pallas import tpu as pltpu


def _gelu(x):
    return x * jax.nn.sigmoid(1.5957691216057308 * x +
                              0.07135481627362622 * (x * x * x))


def _fused_block(x_ref, wpre_ref, wres_ref, wrt_ref, w1_ref, w2_ref,
                 wpost_ref, out_ref, carry_ref, wres16_ref, w116_ref,
                 w216_ref, wpost16_ref, *, lb):
    i = pl.program_id(0)
    f32 = jnp.float32
    bf16 = jnp.bfloat16

    @pl.when(i == 0)
    def _():
        carry_ref[...] = jnp.zeros_like(carry_ref)
        wres16_ref[...] = wres_ref[...].astype(bf16)
        w116_ref[...] = w1_ref[...].astype(bf16)
        w216_ref[...] = w2_ref[...].astype(bf16)
        wpost16_ref[...] = wpost_ref[...].astype(bf16)

    x = x_ref[0]
    h = jax.nn.gelu(jnp.dot(x, wpre_ref[...], preferred_element_type=f32))
    h16 = h.astype(bf16)
    res = jnp.dot(h16, wres16_ref[...], preferred_element_type=f32)
    logits = jnp.dot(h, wrt_ref[...], preferred_element_type=f32)
    probs = jax.nn.sigmoid(logits)

    row = jax.lax.broadcasted_iota(jnp.int32, (lb, 1), 0)
    mask = (probs >= 0.5) | ((row == 0) & (i == 0))

    t16 = jax.nn.gelu(jnp.dot(h16, w116_ref[...],
                              preferred_element_type=f32)).astype(bf16)
    mid = jnp.dot(t16, w216_ref[...], preferred_element_type=f32)
    yg = mid * probs

    rowf = row.astype(f32)
    c_col = jnp.where(mask, rowf, -1.0)
    rowi = jax.lax.broadcasted_iota(jnp.int32, (lb, lb), 0)
    colj = jax.lax.broadcasted_iota(jnp.int32, (lb, lb), 1)
    eye = (rowi == colj).astype(f32)
    c_row = jnp.dot(jnp.ones((1, lb), f32), eye * c_col,
                    preferred_element_type=f32)
    m_mat = jnp.where(colj <= rowi, jnp.broadcast_to(c_row, (lb, lb)), -1.0)
    b_col = jnp.max(m_mat, axis=1, keepdims=True)
    sel = (b_col == colj.astype(f32)).astype(f32)

    carry_row = carry_ref[7:8, :]
    up = jnp.dot(sel, yg, preferred_element_type=f32)
    up = up + jnp.where(b_col < 0.0, carry_row, 0.0)
    carry_ref[...] = up[lb - 8:, :]

    fused16 = (res + up).astype(bf16)
    out_ref[0] = jax.nn.gelu(jnp.dot(fused16, wpost16_ref[...],
                                     preferred_element_type=f32))


def kernel(hidden_states, x_pack_kwargs, W_pre, W_res, w_router, W1, W2,
           W_post):
    del x_pack_kwargs
    B, L, D = hidden_states.shape
    d_ff = W1.shape[1]
    lb = 512
    wrt = w_router.reshape(D, 1)

    grid = (L // lb,)
    full = lambda a: pl.BlockSpec(a.shape, lambda i: (0,) * a.ndim)
    out = pl.pallas_call(
        functools.partial(_fused_block, lb=lb),
        grid=grid,
        in_specs=[
            pl.BlockSpec((1, lb, D), lambda i: (0, i, 0)),
            full(W_pre), full(W_res), full(wrt), full(W1), full(W2),
            full(W_post),
        ],
        out_specs=pl.BlockSpec((1, lb, D), lambda i: (0, i, 0)),
        out_shape=jax.ShapeDtypeStruct((B, L, D), jnp.float32),
        scratch_shapes=[
            pltpu.VMEM((8, D), jnp.float32),
            pltpu.VMEM((D, D), jnp.bfloat16),
            pltpu.VMEM((D, d_ff), jnp.bfloat16),
            pltpu.VMEM((d_ff, D), jnp.bfloat16),
            pltpu.VMEM((D, D), jnp.bfloat16),
        ],
        compiler_params=pltpu.CompilerParams(
            dimension_semantics=("arbitrary",)),
    )(hidden_states, W_pre, W_res, wrt, W1, W2, W_post)
    return out
